# Initial kernel scaffold; baseline (speedup 1.0000x reference)
import jax
import jax.numpy as jnp
from jax.experimental import pallas as pl


def kernel(x, w0, b0, w1, b1, w2, b2, w3, b3, w4, b4, w5, b5, w6, b6, w7, b7, w8, b8, w9, b9, w10, b10, w11, b11, w12, b12, w13, b13):
    raise NotImplementedError("write your pallas kernel here")


# speedup vs baseline: 2.1408x; 2.1408x over previous
import functools

import jax
import jax.numpy as jnp
from jax.experimental import pallas as pl
from jax.experimental.pallas import tpu as pltpu

_VMEM = 56 * 1024 * 1024


def _reflect_pad_hw(a):
    a = jnp.concatenate([a[1:2], a, a[a.shape[0] - 2:a.shape[0] - 1]], axis=0)
    a = jnp.concatenate([a[:, 1:2], a, a[:, a.shape[1] - 2:a.shape[1] - 1]],
                        axis=1)
    return a


def _conv_cat(xp, w_ref, b_ref):
    hp, wp, cin = xp.shape
    h, w = hp - 2, wp - 2
    cout = w_ref.shape[-1]
    xc = jnp.concatenate([xp[:, 0:w], xp[:, 1:w + 1], xp[:, 2:w + 2]],
                         axis=-1)
    acc = jnp.zeros((h * w, cout), jnp.float32)
    for dy in range(3):
        acc = acc + jnp.dot(xc[dy:dy + h].reshape(h * w, 3 * cin),
                            w_ref[dy], preferred_element_type=jnp.float32)
    acc = jnp.maximum(acc + b_ref[...], 0.0)
    return acc.reshape(h, w, cout).astype(jnp.bfloat16)


def _conv_tap(xp, w_ref, b_ref):
    hp, wp, cin = xp.shape
    h, w = hp - 2, wp - 2
    cout = w_ref.shape[-1]
    acc = jnp.zeros((h * w, cout), jnp.float32)
    for dy in range(3):
        for dx in range(3):
            xs = xp[dy:dy + h, dx:dx + w].reshape(h * w, cin)
            acc = acc + jnp.dot(xs, w_ref[dy, dx],
                                preferred_element_type=jnp.float32)
    acc = jnp.maximum(acc + b_ref[...], 0.0)
    return acc.reshape(h, w, cout).astype(jnp.bfloat16)


def _pool_h(a):
    h, w, c = a.shape
    t = a.reshape(h // 2, 2, w, c)
    return jnp.maximum(t[:, 0], t[:, 1])


def _pool_w(a):
    c = a.shape[-1] // 2
    return jnp.maximum(a[..., :c], a[..., c:])


def _stage1_kernel(x_ref, wa_ref, ba_ref, wb_ref, bb_ref, o_ref, *,
                   th, nchunk):
    i = pl.program_id(1)
    a0 = _conv_cat(x_ref[...], wa_ref, ba_ref)
    r0 = jnp.where(i == 0, a0[2:3], a0[0:1])
    rl = jnp.where(i == nchunk - 1, a0[th - 1:th], a0[th + 1:th + 2])
    a0 = jnp.concatenate([r0, a0[1:th + 1], rl], axis=0)
    a0 = jnp.concatenate([a0[:, 1:2], a0, a0[:, -2:-1]], axis=1)
    a1 = _conv_cat(a0, wb_ref, bb_ref)
    o_ref[...] = _pool_h(a1)


def _run_stage1(xp1, wa, ba, wb, bb, nchunk=4):
    n, hp, wp, _ = xp1.shape
    h, w = hp - 2, wp - 2
    th = h // nchunk
    idx = jnp.clip(jnp.arange(th + 4)[None, :] +
                   th * jnp.arange(nchunk)[:, None] - 1, 0, h + 1)
    xch = xp1[:, idx]
    return pl.pallas_call(
        functools.partial(_stage1_kernel, th=th, nchunk=nchunk),
        out_shape=jax.ShapeDtypeStruct((n, h // 2, w, 64), jnp.bfloat16),
        grid=(n, nchunk),
        in_specs=[
            pl.BlockSpec((None, None, th + 4, wp, 3),
                         lambda b, i: (b, i, 0, 0, 0)),
            pl.BlockSpec((3, 9, 64), lambda b, i: (0, 0, 0)),
            pl.BlockSpec((1, 64), lambda b, i: (0, 0)),
            pl.BlockSpec((3, 192, 64), lambda b, i: (0, 0, 0)),
            pl.BlockSpec((1, 64), lambda b, i: (0, 0)),
        ],
        out_specs=pl.BlockSpec((None, th // 2, w, 64),
                               lambda b, i: (b, i, 0, 0)),
        compiler_params=pltpu.CompilerParams(
            dimension_semantics=("parallel", "parallel"),
            vmem_limit_bytes=_VMEM),
    )(xch, wa, ba, wb, bb)


def _stage_kernel(*refs, modes, pool, out_f32):
    x_ref = refs[0]
    o_ref = refs[-1]
    a = _pool_w(x_ref[...])
    for k, mode in enumerate(modes):
        w_ref, b_ref = refs[1 + 2 * k], refs[2 + 2 * k]
        ap = _reflect_pad_hw(a)
        a = _conv_cat(ap, w_ref, b_ref) if mode == 'c' else \
            _conv_tap(ap, w_ref, b_ref)
    if pool:
        a = _pool_h(a)
    o_ref[...] = a.astype(jnp.float32) if out_f32 else a


def _run_stage(x, wbs, modes, pool, out_f32=False):
    n, h, w, _ = x.shape
    cout = wbs[-1][0].shape[-1]
    ho, wo = (h // 2, w) if pool else (h, w)
    ins = [x]
    in_specs = [pl.BlockSpec((None,) + x.shape[1:],
                             lambda b: (b, 0, 0, 0))]
    for wt, bt in wbs:
        ins += [wt, bt]
        nd = len(wt.shape)
        in_specs.append(pl.BlockSpec(wt.shape, lambda b, _nd=nd: (0,) * _nd))
        in_specs.append(pl.BlockSpec(bt.shape, lambda b: (0, 0)))
    return pl.pallas_call(
        functools.partial(_stage_kernel, modes=modes, pool=pool,
                          out_f32=out_f32),
        out_shape=jax.ShapeDtypeStruct(
            (n, ho, wo, cout), jnp.float32 if out_f32 else jnp.bfloat16),
        grid=(n,),
        in_specs=in_specs,
        out_specs=pl.BlockSpec((None, ho, wo, cout), lambda b: (b, 0, 0, 0)),
        compiler_params=pltpu.CompilerParams(
            dimension_semantics=("parallel",),
            vmem_limit_bytes=_VMEM),
    )(*ins)


def _prep_cat(w, b):
    k, _, cin, cout = w.shape
    return (w.astype(jnp.bfloat16).reshape(k, k * cin, cout),
            b.reshape(1, cout).astype(jnp.float32))


def _prep_tap(w, b):
    return (w.astype(jnp.bfloat16),
            b.reshape(1, w.shape[-1]).astype(jnp.float32))


def kernel(x, w0, b0, w1, b1, w2, b2, w3, b3, w4, b4, w5, b5, w6, b6,
           w7, b7, w8, b8, w9, b9, w10, b10, w11, b11, w12, b12, w13, b13):
    wa = jnp.einsum('im,klmo->klio', w0[0, 0], w1)
    ba = b1 + jnp.einsum('klmo,m->o', w1, b0)

    xb = x.astype(jnp.bfloat16)
    xp1 = jnp.pad(xb, ((0, 0), (1, 1), (1, 1), (0, 0)), mode='reflect')

    def wfold(a):
        n_, h_, w_, c_ = a.shape
        return a.reshape(n_, h_, w_ // 2, 2 * c_)

    a = _run_stage1(xp1, *_prep_cat(wa, ba), *_prep_cat(w2, b2))
    a = _run_stage(wfold(a), [_prep_cat(w3, b3), _prep_cat(w4, b4)],
                   modes='cc', pool=True)
    a = _run_stage(wfold(a), [_prep_cat(w5, b5), _prep_tap(w6, b6),
                              _prep_tap(w7, b7), _prep_tap(w8, b8)],
                   modes='cttt', pool=True)
    a = _run_stage(wfold(a), [_prep_tap(w9, b9), _prep_tap(w10, b10),
                              _prep_tap(w11, b11), _prep_tap(w12, b12)],
                   modes='tttt', pool=True)
    a = _run_stage(wfold(a), [_prep_tap(w13, b13)], modes='t', pool=False,
                   out_f32=True)
    return jnp.transpose(a, (0, 3, 1, 2))


# speedup vs baseline: 2.1760x; 1.0165x over previous
import functools

import jax
import jax.numpy as jnp
from jax.experimental import pallas as pl
from jax.experimental.pallas import tpu as pltpu

_VMEM = 56 * 1024 * 1024


def _reflect_pad_hw(a):
    a = jnp.concatenate([a[1:2], a, a[a.shape[0] - 2:a.shape[0] - 1]], axis=0)
    a = jnp.concatenate([a[:, 1:2], a, a[:, a.shape[1] - 2:a.shape[1] - 1]],
                        axis=1)
    return a


def _conv_cat(xp, w_ref, b_ref):
    hp, wp, cin = xp.shape
    h, w = hp - 2, wp - 2
    cout = w_ref.shape[-1]
    xc = jnp.concatenate([xp[:, 0:w], xp[:, 1:w + 1], xp[:, 2:w + 2]],
                         axis=-1)
    acc = jnp.zeros((h * w, cout), jnp.float32)
    for dy in range(3):
        acc = acc + jnp.dot(xc[dy:dy + h].reshape(h * w, 3 * cin),
                            w_ref[dy], preferred_element_type=jnp.float32)
    acc = jnp.maximum(acc + b_ref[...], 0.0)
    return acc.reshape(h, w, cout).astype(jnp.bfloat16)


def _conv_tap(xp, w_ref, b_ref):
    hp, wp, cin = xp.shape
    h, w = hp - 2, wp - 2
    cout = w_ref.shape[-1]
    acc = jnp.zeros((h * w, cout), jnp.float32)
    for dy in range(3):
        for dx in range(3):
            xs = xp[dy:dy + h, dx:dx + w].reshape(h * w, cin)
            acc = acc + jnp.dot(xs, w_ref[dy, dx],
                                preferred_element_type=jnp.float32)
    acc = jnp.maximum(acc + b_ref[...], 0.0)
    return acc.reshape(h, w, cout).astype(jnp.bfloat16)


def _pool_h(a):
    h, w, c = a.shape
    t = a.reshape(h // 2, 2, w, c)
    return jnp.maximum(t[:, 0], t[:, 1])


def _pool_w(a):
    c = a.shape[-1] // 2
    return jnp.maximum(a[..., :c], a[..., c:])


def _stage1_kernel(x_ref, wa_ref, ba_ref, wb_ref, bb_ref, o_ref, *,
                   th, nchunk):
    i = pl.program_id(1)
    a0 = _conv_cat(x_ref[...], wa_ref, ba_ref)
    r0 = jnp.where(i == 0, a0[2:3], a0[0:1])
    rl = jnp.where(i == nchunk - 1, a0[th - 1:th], a0[th + 1:th + 2])
    a0 = jnp.concatenate([r0, a0[1:th + 1], rl], axis=0)
    a0 = jnp.concatenate([a0[:, 1:2], a0, a0[:, -2:-1]], axis=1)
    a1 = _conv_cat(a0, wb_ref, bb_ref)
    o_ref[...] = _pool_h(a1)


def _run_stage1(xp1, wa, ba, wb, bb, nchunk=4):
    n, hp, wp, _ = xp1.shape
    h, w = hp - 2, wp - 2
    th = h // nchunk
    parts = []
    for i in range(nchunk):
        lo = th * i - 1
        hi = lo + th + 4
        seg = xp1[:, max(lo, 0):min(hi, h + 2)]
        if lo < 0:
            seg = jnp.concatenate([seg[:, :1]] * (-lo) + [seg], axis=1)
        if hi > h + 2:
            seg = jnp.concatenate([seg] + [seg[:, -1:]] * (hi - h - 2), axis=1)
        parts.append(seg)
    xch = jnp.stack(parts, axis=1)
    return pl.pallas_call(
        functools.partial(_stage1_kernel, th=th, nchunk=nchunk),
        out_shape=jax.ShapeDtypeStruct((n, h // 2, w, 64), jnp.bfloat16),
        grid=(n, nchunk),
        in_specs=[
            pl.BlockSpec((None, None, th + 4, wp, 3),
                         lambda b, i: (b, i, 0, 0, 0)),
            pl.BlockSpec((3, 9, 64), lambda b, i: (0, 0, 0)),
            pl.BlockSpec((1, 64), lambda b, i: (0, 0)),
            pl.BlockSpec((3, 192, 64), lambda b, i: (0, 0, 0)),
            pl.BlockSpec((1, 64), lambda b, i: (0, 0)),
        ],
        out_specs=pl.BlockSpec((None, th // 2, w, 64),
                               lambda b, i: (b, i, 0, 0)),
        compiler_params=pltpu.CompilerParams(
            dimension_semantics=("parallel", "parallel"),
            vmem_limit_bytes=_VMEM),
    )(xch, wa, ba, wb, bb)


def _stage_kernel(*refs, modes, pool, out_f32):
    x_ref = refs[0]
    o_ref = refs[-1]
    a = _pool_w(x_ref[...])
    for k, mode in enumerate(modes):
        w_ref, b_ref = refs[1 + 2 * k], refs[2 + 2 * k]
        ap = _reflect_pad_hw(a)
        a = _conv_cat(ap, w_ref, b_ref) if mode == 'c' else \
            _conv_tap(ap, w_ref, b_ref)
    if pool:
        a = _pool_h(a)
    o_ref[...] = a.astype(jnp.float32) if out_f32 else a


def _run_stage(x, wbs, modes, pool, out_f32=False):
    n, h, w, _ = x.shape
    cout = wbs[-1][0].shape[-1]
    ho, wo = (h // 2, w) if pool else (h, w)
    ins = [x]
    in_specs = [pl.BlockSpec((None,) + x.shape[1:],
                             lambda b: (b, 0, 0, 0))]
    for wt, bt in wbs:
        ins += [wt, bt]
        nd = len(wt.shape)
        in_specs.append(pl.BlockSpec(wt.shape, lambda b, _nd=nd: (0,) * _nd))
        in_specs.append(pl.BlockSpec(bt.shape, lambda b: (0, 0)))
    return pl.pallas_call(
        functools.partial(_stage_kernel, modes=modes, pool=pool,
                          out_f32=out_f32),
        out_shape=jax.ShapeDtypeStruct(
            (n, ho, wo, cout), jnp.float32 if out_f32 else jnp.bfloat16),
        grid=(n,),
        in_specs=in_specs,
        out_specs=pl.BlockSpec((None, ho, wo, cout), lambda b: (b, 0, 0, 0)),
        compiler_params=pltpu.CompilerParams(
            dimension_semantics=("parallel",),
            vmem_limit_bytes=_VMEM),
    )(*ins)


def _prep_cat(w, b):
    k, _, cin, cout = w.shape
    return (w.astype(jnp.bfloat16).reshape(k, k * cin, cout),
            b.reshape(1, cout).astype(jnp.float32))


def _prep_tap(w, b):
    return (w.astype(jnp.bfloat16),
            b.reshape(1, w.shape[-1]).astype(jnp.float32))


def kernel(x, w0, b0, w1, b1, w2, b2, w3, b3, w4, b4, w5, b5, w6, b6,
           w7, b7, w8, b8, w9, b9, w10, b10, w11, b11, w12, b12, w13, b13):
    wa = jnp.einsum('im,klmo->klio', w0[0, 0], w1)
    ba = b1 + jnp.einsum('klmo,m->o', w1, b0)

    xb = x.astype(jnp.bfloat16)
    xp1 = jnp.pad(xb, ((0, 0), (1, 1), (1, 1), (0, 0)), mode='reflect')

    def wfold(a):
        n_, h_, w_, c_ = a.shape
        return a.reshape(n_, h_, w_ // 2, 2 * c_)

    a = _run_stage1(xp1, *_prep_cat(wa, ba), *_prep_cat(w2, b2))
    a = _run_stage(wfold(a), [_prep_cat(w3, b3), _prep_cat(w4, b4)],
                   modes='cc', pool=True)
    a = _run_stage(wfold(a), [_prep_cat(w5, b5), _prep_cat(w6, b6),
                              _prep_cat(w7, b7), _prep_cat(w8, b8)],
                   modes='cccc', pool=True)
    a = _run_stage(wfold(a), [_prep_cat(w9, b9), _prep_cat(w10, b10),
                              _prep_cat(w11, b11), _prep_cat(w12, b12)],
                   modes='cccc', pool=True)
    a = _run_stage(wfold(a), [_prep_cat(w13, b13)], modes='c', pool=False,
                   out_f32=True)
    return jnp.transpose(a, (0, 3, 1, 2))
